# baseline (speedup 1.0000x reference)
import jax
import jax.numpy as jnp
from jax.experimental import pallas as pl
from jax.experimental.pallas import tpu as pltpu


def _copy_body(k_ref, v_ref, ok_ref, ov_ref, sem_k, sem_v):
    ck = pltpu.make_async_copy(k_ref, ok_ref, sem_k)
    cv = pltpu.make_async_copy(v_ref, ov_ref, sem_v)
    ck.start()
    cv.start()
    ck.wait()
    cv.wait()


def kernel(new_k, new_v):
    out_shape = (
        jax.ShapeDtypeStruct(new_k.shape, new_k.dtype),
        jax.ShapeDtypeStruct(new_v.shape, new_v.dtype),
    )
    return pl.pallas_call(
        _copy_body,
        out_shape=out_shape,
        in_specs=[
            pl.BlockSpec(memory_space=pl.ANY),
            pl.BlockSpec(memory_space=pl.ANY),
        ],
        out_specs=[
            pl.BlockSpec(memory_space=pl.ANY),
            pl.BlockSpec(memory_space=pl.ANY),
        ],
        scratch_shapes=[pltpu.SemaphoreType.DMA, pltpu.SemaphoreType.DMA],
    )(new_k, new_v)


# speedup vs baseline: 1.0004x; 1.0004x over previous
import jax
import jax.numpy as jnp
from jax.experimental import pallas as pl
from jax.experimental.pallas import tpu as pltpu

_CHUNKS = 16


def _copy_body(k_ref, v_ref, ok_ref, ov_ref, sems):
    n = k_ref.shape[0]
    ch = n // _CHUNKS
    copies = []
    for i in range(_CHUNKS):
        sl = pl.ds(i * ch, ch)
        copies.append(
            pltpu.make_async_copy(k_ref.at[sl], ok_ref.at[sl], sems.at[i])
        )
        copies.append(
            pltpu.make_async_copy(v_ref.at[sl], ov_ref.at[sl], sems.at[_CHUNKS + i])
        )
    for c in copies:
        c.start()
    for c in copies:
        c.wait()


def kernel(new_k, new_v):
    B, H, L, HD = new_k.shape
    k2 = new_k.reshape(B * H * L, HD)
    v2 = new_v.reshape(B * H * L, HD)
    out_shape = (
        jax.ShapeDtypeStruct(k2.shape, k2.dtype),
        jax.ShapeDtypeStruct(v2.shape, v2.dtype),
    )
    ok, ov = pl.pallas_call(
        _copy_body,
        out_shape=out_shape,
        in_specs=[
            pl.BlockSpec(memory_space=pl.ANY),
            pl.BlockSpec(memory_space=pl.ANY),
        ],
        out_specs=[
            pl.BlockSpec(memory_space=pl.ANY),
            pl.BlockSpec(memory_space=pl.ANY),
        ],
        scratch_shapes=[pltpu.SemaphoreType.DMA((2 * _CHUNKS,))],
    )(k2, v2)
    return ok.reshape(B, H, L, HD), ov.reshape(B, H, L, HD)


# speedup vs baseline: 47.9983x; 47.9812x over previous
import jax
import jax.numpy as jnp
from jax.experimental import pallas as pl
from jax.experimental.pallas import tpu as pltpu

_GRID = 64


def _copy_body(k_ref, v_ref, ok_ref, ov_ref):
    ok_ref[...] = k_ref[...]
    ov_ref[...] = v_ref[...]


def kernel(new_k, new_v):
    B, H, L, HD = new_k.shape
    rows = B * H * L // _GRID
    k2 = new_k.reshape(_GRID, rows, HD)
    v2 = new_v.reshape(_GRID, rows, HD)
    out_shape = (
        jax.ShapeDtypeStruct(k2.shape, k2.dtype),
        jax.ShapeDtypeStruct(v2.shape, v2.dtype),
    )
    spec = pl.BlockSpec((1, rows, HD), lambda i: (i, 0, 0))
    ok, ov = pl.pallas_call(
        _copy_body,
        grid=(_GRID,),
        out_shape=out_shape,
        in_specs=[spec, spec],
        out_specs=[spec, spec],
    )(k2, v2)
    return ok.reshape(B, H, L, HD), ov.reshape(B, H, L, HD)
